# Initial kernel scaffold; baseline (speedup 1.0000x reference)
import jax
import jax.numpy as jnp
from jax.experimental import pallas as pl


def kernel(table, u_pos, v_pos, v_neg):
    raise NotImplementedError("write your pallas kernel here")


# speedup vs baseline: 8.6034x; 8.6034x over previous
import jax
import jax.numpy as jnp
from jax import lax
from jax.experimental import pallas as pl
from jax.experimental.pallas import tpu as pltpu
from jax.experimental.pallas import tpu_sc as plsc

D = 64
P = 20
N = 50
B = 4096
NC, NS = 2, 16
NW = NC * NS
BPW = B // NW
E = 16
NCH = BPW // E
G = 80
GP = (E * P) // G
GN = (E * N) // G
LANES = 16
KD = D // LANES


def _sc_body(table, u_idx, p_idx, n_idx, sc_out, nsc_out,
             idx_u, idx_p, idx_n, u_rows, ring, s_v, n_v,
             sem_u, sem_g):
    wid = lax.axis_index("s") * NC + lax.axis_index("c")
    pltpu.sync_copy(u_idx.at[wid], idx_u)
    pltpu.sync_copy(p_idx.at[wid], idx_p)
    pltpu.sync_copy(n_idx.at[wid], idx_n)
    pltpu.async_copy(table.at[idx_u], u_rows, sem_u).wait()

    def _dot_u(buf, row, r0, stride):
        acc = [jnp.zeros((LANES,), jnp.float32) for _ in range(KD)]
        for j in range(stride):
            r = r0 + j
            for k in range(KD):
                acc[k] = acc[k] + buf[r, pl.ds(LANES * k, LANES)]
        dp = acc[0] * u_rows[row, pl.ds(0, LANES)]
        for k in range(1, KD):
            dp = dp + acc[k] * u_rows[row, pl.ds(LANES * k, LANES)]
        return dp

    @pl.loop(0, NCH)
    def _pos_chunk(c):
        cps = [pltpu.async_copy(table.at[idx_p.at[GP * c + g]],
                                ring.at[0, pl.ds(G * g, G)], sem_g)
               for g in range(GP)]
        for cp in cps:
            cp.wait()

        @pl.loop(0, E)
        def _elem(e):
            row = c * E + e
            s_v[row, pl.ds(0, LANES)] = _dot_u(ring.at[0], row, e * P, P)

    @pl.loop(0, NCH)
    def _neg_chunk(c):
        cps = [pltpu.async_copy(table.at[idx_n.at[GN * c + g]],
                                ring.at[0, pl.ds(G * g, G)], sem_g)
               for g in range(GN)]
        for cp in cps:
            cp.wait()

        @pl.loop(0, E)
        def _elem(e):
            row = c * E + e
            n_v[row, pl.ds(0, LANES)] = _dot_u(ring.at[0], row, e * N, N)

    pltpu.sync_copy(s_v, sc_out.at[pl.ds(wid * BPW, BPW)])
    pltpu.sync_copy(n_v, nsc_out.at[pl.ds(wid * BPW, BPW)])


_sc_scores = pl.kernel(
    _sc_body,
    out_type=(jax.ShapeDtypeStruct((B, LANES), jnp.float32),
              jax.ShapeDtypeStruct((B, LANES), jnp.float32)),
    mesh=plsc.VectorSubcoreMesh(core_axis_name="c", subcore_axis_name="s",
                                num_cores=NC, num_subcores=NS),
    scratch_types=[
        pltpu.VMEM((BPW,), jnp.int32),
        pltpu.VMEM((BPW * P // G, G), jnp.int32),
        pltpu.VMEM((BPW * N // G, G), jnp.int32),
        pltpu.VMEM((BPW, D), jnp.float32),
        pltpu.VMEM((2, E * N, D), jnp.float32),
        pltpu.VMEM((BPW, LANES), jnp.float32),
        pltpu.VMEM((BPW, LANES), jnp.float32),
        pltpu.SemaphoreType.DMA,
        pltpu.SemaphoreType.DMA,
    ],
    compiler_params=pltpu.CompilerParams(use_tc_tiling_on_sc=False),
)


def _loss_body(s_ref, n_ref, o_ref):
    s = jnp.sum(s_ref[...], axis=1) * (1.0 / P)
    n = jnp.sum(n_ref[...], axis=1) * (-1.0 / N)
    ls = jnp.minimum(s, 0.0) - jnp.log(1.0 + jnp.exp(-jnp.abs(s)))
    ln = jnp.minimum(n, 0.0) - jnp.log(1.0 + jnp.exp(-jnp.abs(n)))
    o_ref[0, 0] = -(jnp.sum(ls) + jnp.sum(ln)) / B


_loss = pl.pallas_call(
    _loss_body,
    out_shape=jax.ShapeDtypeStruct((1, 1), jnp.float32),
    in_specs=[pl.BlockSpec(memory_space=pltpu.VMEM),
              pl.BlockSpec(memory_space=pltpu.VMEM)],
    out_specs=pl.BlockSpec(memory_space=pltpu.SMEM),
)


def kernel(table, u_pos, v_pos, v_neg):
    u_idx = u_pos.reshape(NW, BPW)
    p_idx = v_pos.reshape(NW, BPW * P // G, G)
    n_idx = v_neg.reshape(NW, BPW * N // G, G)
    scores, neg_scores = _sc_scores(table, u_idx, p_idx, n_idx)
    return _loss(scores, neg_scores)[0, 0]


# speedup vs baseline: 9.7694x; 1.1355x over previous
import jax
import jax.numpy as jnp
from jax import lax
from jax.experimental import pallas as pl
from jax.experimental.pallas import tpu as pltpu
from jax.experimental.pallas import tpu_sc as plsc

D = 64
P = 20
N = 50
B = 4096
NC, NS = 2, 16
NW = NC * NS
BPW = B // NW
E = 16
NCH = BPW // E
G = 80
GP = (E * P) // G
GN = (E * N) // G
LANES = 16
KD = D // LANES


def _sc_body(table, u_idx, p_idx, n_idx, sc_out, nsc_out,
             idx_u, idx_p, idx_n, u_rows, ring, s_v, n_v,
             sem_u, sem_g0, sem_g1):
    wid = lax.axis_index("s") * NC + lax.axis_index("c")
    pltpu.sync_copy(u_idx.at[wid], idx_u)
    pltpu.sync_copy(p_idx.at[wid], idx_p)
    pltpu.sync_copy(n_idx.at[wid], idx_n)
    pltpu.async_copy(table.at[idx_u], u_rows, sem_u).wait()

    def _dot_u(buf, row, r0, stride):
        acc = [jnp.zeros((LANES,), jnp.float32) for _ in range(KD)]
        for j in range(stride):
            r = r0 + j
            for k in range(KD):
                acc[k] = acc[k] + buf[r, pl.ds(LANES * k, LANES)]
        dp = acc[0] * u_rows[row, pl.ds(0, LANES)]
        for k in range(1, KD):
            dp = dp + acc[k] * u_rows[row, pl.ds(LANES * k, LANES)]
        return dp

    def _pipelined_pass(idx2d, gpc, rows_per_e, out_v):
        def fire(c, par):
            for g in range(gpc):
                pltpu.async_copy(table.at[idx2d.at[gpc * c + g]],
                                 ring.at[par, pl.ds(G * g, G)], sems[par])

        def wait(c, par):
            for g in range(gpc):
                pltpu.make_async_copy(table.at[idx2d.at[gpc * c + g]],
                                      ring.at[par, pl.ds(G * g, G)],
                                      sems[par]).wait()

        def compute(c, par):
            @pl.loop(0, E)
            def _elem(e):
                row = c * E + e
                out_v[row, pl.ds(0, LANES)] = _dot_u(
                    ring.at[par], row, e * rows_per_e, rows_per_e)

        fire(0, 0)
        fire(1, 1)

        @pl.loop(0, NCH - 2, step=2)
        def _steady(c0):
            for par in (0, 1):
                c = c0 + par
                wait(c, par)
                compute(c, par)
                fire(c + 2, par)

        for par in (0, 1):
            c = NCH - 2 + par
            wait(c, par)
            compute(c, par)

    sems = (sem_g0, sem_g1)
    _pipelined_pass(idx_p, GP, P, s_v)
    _pipelined_pass(idx_n, GN, N, n_v)

    pltpu.sync_copy(s_v, sc_out.at[pl.ds(wid * BPW, BPW)])
    pltpu.sync_copy(n_v, nsc_out.at[pl.ds(wid * BPW, BPW)])


_sc_scores = pl.kernel(
    _sc_body,
    out_type=(jax.ShapeDtypeStruct((B, LANES), jnp.float32),
              jax.ShapeDtypeStruct((B, LANES), jnp.float32)),
    mesh=plsc.VectorSubcoreMesh(core_axis_name="c", subcore_axis_name="s",
                                num_cores=NC, num_subcores=NS),
    scratch_types=[
        pltpu.VMEM((BPW,), jnp.int32),
        pltpu.VMEM((BPW * P // G, G), jnp.int32),
        pltpu.VMEM((BPW * N // G, G), jnp.int32),
        pltpu.VMEM((BPW, D), jnp.float32),
        pltpu.VMEM((2, E * N, D), jnp.float32),
        pltpu.VMEM((BPW, LANES), jnp.float32),
        pltpu.VMEM((BPW, LANES), jnp.float32),
        pltpu.SemaphoreType.DMA,
        pltpu.SemaphoreType.DMA,
        pltpu.SemaphoreType.DMA,
    ],
    compiler_params=pltpu.CompilerParams(use_tc_tiling_on_sc=False),
)


def _loss_body(s_ref, n_ref, o_ref):
    s = jnp.sum(s_ref[...], axis=1) * (1.0 / P)
    n = jnp.sum(n_ref[...], axis=1) * (-1.0 / N)
    ls = jnp.minimum(s, 0.0) - jnp.log(1.0 + jnp.exp(-jnp.abs(s)))
    ln = jnp.minimum(n, 0.0) - jnp.log(1.0 + jnp.exp(-jnp.abs(n)))
    o_ref[0, 0] = -(jnp.sum(ls) + jnp.sum(ln)) / B


_loss = pl.pallas_call(
    _loss_body,
    out_shape=jax.ShapeDtypeStruct((1, 1), jnp.float32),
    in_specs=[pl.BlockSpec(memory_space=pltpu.VMEM),
              pl.BlockSpec(memory_space=pltpu.VMEM)],
    out_specs=pl.BlockSpec(memory_space=pltpu.SMEM),
)


def kernel(table, u_pos, v_pos, v_neg):
    u_idx = u_pos.reshape(NW, BPW)
    p_idx = v_pos.reshape(NW, BPW * P // G, G)
    n_idx = v_neg.reshape(NW, BPW * N // G, G)
    scores, neg_scores = _sc_scores(table, u_idx, p_idx, n_idx)
    return _loss(scores, neg_scores)[0, 0]
